# Initial kernel scaffold; baseline (speedup 1.0000x reference)
import jax
import jax.numpy as jnp
from jax.experimental import pallas as pl


def kernel(x, edge_index, W1_self, b1_self, W1_neigh, b1_neigh, W2_self, b2_self, W2_neigh, b2_neigh, Wc, bc):
    raise NotImplementedError("write your pallas kernel here")


# speedup vs baseline: 7.4030x; 7.4030x over previous
import functools

import jax
import jax.numpy as jnp
from jax import lax
from jax.experimental import pallas as pl
from jax.experimental.pallas import tpu as pltpu
from jax.experimental.pallas import tpu_sc as plsc

N = 10000
E = 320000
DIN = 128
HID = 256
DOUT = 64

EPAD = 327680

CH1 = 64
K1 = EPAD // (16 * CH1)
SB1 = 64
NROW1 = EPAD // CH1

CH2 = 128
K2 = EPAD // (16 * CH2)
SB2 = 32
NROW2 = EPAD // CH2

HALF = N // 2
R1 = 5120
RPS1 = R1 // 16

ACC2 = 10112
RPS2 = ACC2 // 16

_f32 = jnp.float32
_mesh = plsc.VectorSubcoreMesh(core_axis_name="c", subcore_axis_name="s")


@functools.partial(
    pl.kernel,
    out_type=[jax.ShapeDtypeStruct((2, R1, DIN), _f32),
              jax.ShapeDtypeStruct((2, R1, DIN), _f32)],
    mesh=_mesh,
    scratch_types=[
        pltpu.VMEM((SB1, CH1), jnp.int32),
        pltpu.VMEM((SB1, CH1), jnp.int32),
        pltpu.VMEM((CH1, DIN), _f32),
        pltpu.VMEM((CH1, DIN), _f32),
        pltpu.VMEM((16, DIN), _f32),
        pltpu.VMEM((CH1, DIN), _f32),
        pltpu.VMEM_SHARED((R1, DIN), _f32),
        pltpu.VMEM_SHARED((R1, DIN), _f32),
        pltpu.SemaphoreType.DMA,
        pltpu.SemaphoreType.DMA,
    ],
)
def _sc_pass1(x_hbm, src_hbm, dst_hbm, agg_hbm, deg_hbm,
              sidx, didx, rows0, rows1, zbuf, ones, acc, dega,
              sem0, sem1):
    c = lax.axis_index("c")
    s = lax.axis_index("s")

    zv = jnp.zeros((16,), _f32)
    ov = jnp.ones((16,), _f32)

    @pl.loop(0, 16)
    def _(r):
        @pl.loop(0, DIN // 16)
        def _(q):
            zbuf[r, pl.ds(q * 16, 16)] = zv

    @pl.loop(0, CH1)
    def _(r):
        @pl.loop(0, DIN // 16)
        def _(q):
            ones[r, pl.ds(q * 16, 16)] = ov

    @pl.loop(0, RPS1 // 16)
    def _(k):
        r0 = s * RPS1 + k * 16
        pltpu.sync_copy(zbuf, acc.at[pl.ds(r0, 16)])
        pltpu.sync_copy(zbuf, dega.at[pl.ds(r0, 16)])

    plsc.subcore_barrier()

    @pl.loop(0, K1 // SB1)
    def _(g):
        b0 = s * K1 + g * SB1
        pltpu.sync_copy(src_hbm.at[pl.ds(b0, SB1)], sidx)
        pltpu.sync_copy(dst_hbm.at[c, pl.ds(b0, SB1)], didx)

        pltpu.async_copy(x_hbm.at[sidx.at[0]], rows0, sem0)

        @pl.loop(0, SB1, step=2)
        def _(j):
            pltpu.make_async_copy(x_hbm.at[sidx.at[j]], rows0, sem0).wait()
            pltpu.async_copy(x_hbm.at[sidx.at[j + 1]], rows1, sem1)
            pltpu.sync_copy(rows0, acc.at[didx.at[j]], add=True)
            pltpu.sync_copy(ones, dega.at[didx.at[j]], add=True)
            pltpu.make_async_copy(x_hbm.at[sidx.at[j + 1]],
                                  rows1, sem1).wait()

            @pl.when(j + 2 < SB1)
            def _():
                pltpu.async_copy(x_hbm.at[sidx.at[j + 2]], rows0, sem0)

            pltpu.sync_copy(rows1, acc.at[didx.at[j + 1]], add=True)
            pltpu.sync_copy(ones, dega.at[didx.at[j + 1]], add=True)

    plsc.subcore_barrier()

    r0 = s * RPS1
    pltpu.sync_copy(acc.at[pl.ds(r0, RPS1)], agg_hbm.at[c, pl.ds(r0, RPS1)])
    pltpu.sync_copy(dega.at[pl.ds(r0, RPS1)], deg_hbm.at[c, pl.ds(r0, RPS1)])


@functools.partial(
    pl.kernel,
    out_type=jax.ShapeDtypeStruct((2, ACC2, DIN), _f32),
    mesh=_mesh,
    scratch_types=[
        pltpu.VMEM((SB2, CH2), jnp.int32),
        pltpu.VMEM((SB2, CH2), jnp.int32),
        pltpu.VMEM((CH2, DIN), _f32),
        pltpu.VMEM((CH2, DIN), _f32),
        pltpu.VMEM((16, DIN), _f32),
        pltpu.VMEM_SHARED((ACC2, DIN), _f32),
        pltpu.SemaphoreType.DMA,
        pltpu.SemaphoreType.DMA,
    ],
)
def _sc_pass2(p_hbm, src_hbm, dst_hbm, out_hbm,
              sidx, didx, rows0, rows1, zbuf, acc, sem0, sem1):
    c = lax.axis_index("c")
    s = lax.axis_index("s")

    zv = jnp.zeros((16,), _f32)

    @pl.loop(0, 16)
    def _(r):
        @pl.loop(0, DIN // 16)
        def _(q):
            zbuf[r, pl.ds(q * 16, 16)] = zv

    @pl.loop(0, RPS2 // 16)
    def _(k):
        pltpu.sync_copy(zbuf, acc.at[pl.ds(s * RPS2 + k * 16, 16)])

    pltpu.sync_copy(zbuf.at[pl.ds(0, RPS2 % 16)],
                    acc.at[pl.ds(s * RPS2 + (RPS2 // 16) * 16, RPS2 % 16)])

    plsc.subcore_barrier()

    @pl.loop(0, K2 // SB2)
    def _(g):
        b0 = s * K2 + g * SB2
        pltpu.sync_copy(src_hbm.at[c, pl.ds(b0, SB2)], sidx)
        pltpu.sync_copy(dst_hbm.at[pl.ds(b0, SB2)], didx)

        pltpu.async_copy(p_hbm.at[sidx.at[0]], rows0, sem0)

        @pl.loop(0, SB2, step=2)
        def _(j):
            pltpu.make_async_copy(p_hbm.at[sidx.at[j]], rows0, sem0).wait()
            pltpu.async_copy(p_hbm.at[sidx.at[j + 1]], rows1, sem1)
            pltpu.sync_copy(rows0, acc.at[didx.at[j]], add=True)
            pltpu.make_async_copy(p_hbm.at[sidx.at[j + 1]],
                                  rows1, sem1).wait()

            @pl.when(j + 2 < SB2)
            def _():
                pltpu.async_copy(p_hbm.at[sidx.at[j + 2]], rows0, sem0)

            pltpu.sync_copy(rows1, acc.at[didx.at[j + 1]], add=True)

    plsc.subcore_barrier()

    r0 = s * RPS2
    pltpu.sync_copy(acc.at[pl.ds(r0, RPS2)], out_hbm.at[c, pl.ds(r0, RPS2)])


BM = 1000


def _tc_phase_b(x, aggp, degp, W1s, b1s, W1n, b1n, W2s, W2n):
    def body(x_ref, aggp_ref, degp_ref, w1s_ref, b1s_ref, w1n_ref, b1n_ref,
             w2s_ref, w2n_ref, q_ref, pcat_ref):
        deg = degp_ref[0, :, 0:1]
        inv = 1.0 / jnp.maximum(deg, 1.0)
        agg = aggp_ref[0] * inv
        hs = jnp.dot(x_ref[...], w1s_ref[...],
                     preferred_element_type=_f32) + b1s_ref[...]
        hn = jnp.dot(agg, w1n_ref[...],
                     preferred_element_type=_f32) + b1n_ref[...]
        h = jnp.maximum(jnp.concatenate([hs, hn], axis=1), 0.0)
        q_ref[...] = jnp.dot(h, w2s_ref[...], preferred_element_type=_f32)
        p = jnp.dot(h, w2n_ref[...], preferred_element_type=_f32)
        pcat_ref[0, :, :] = p[:, :DIN]
        pcat_ref[1, :, :] = p[:, DIN:]

    return pl.pallas_call(
        body,
        grid=(N // BM,),
        in_specs=[
            pl.BlockSpec((BM, DIN), lambda i: (i, 0)),
            pl.BlockSpec((1, BM, DIN), lambda i: (i // 5, i % 5, 0)),
            pl.BlockSpec((1, BM, DIN), lambda i: (i // 5, i % 5, 0)),
            pl.BlockSpec((DIN, HID), lambda i: (0, 0)),
            pl.BlockSpec((HID,), lambda i: (0,)),
            pl.BlockSpec((DIN, HID), lambda i: (0, 0)),
            pl.BlockSpec((HID,), lambda i: (0,)),
            pl.BlockSpec((2 * HID, HID), lambda i: (0, 0)),
            pl.BlockSpec((2 * HID, HID), lambda i: (0, 0)),
        ],
        out_specs=[
            pl.BlockSpec((BM, HID), lambda i: (i, 0)),
            pl.BlockSpec((2, BM, DIN), lambda i: (0, i, 0)),
        ],
        out_shape=[jax.ShapeDtypeStruct((N, HID), _f32),
                   jax.ShapeDtypeStruct((2, N, DIN), _f32)],
    )(x, aggp, degp, W1s, b1s, W1n, b1n, W2s, W2n)


def _tc_phase_d(q, agg2, degp, b2s, b2n, Wc, bc):
    def body(q_ref, a2_ref, degp_ref, b2s_ref, b2n_ref, wc_ref, bc_ref,
             o_ref):
        deg = degp_ref[0, :, 0:1]
        inv = 1.0 / jnp.maximum(deg, 1.0)
        hs = q_ref[...] + b2s_ref[...]
        a2 = jnp.concatenate([a2_ref[0], a2_ref[1]], axis=1) * inv
        h = jnp.maximum(jnp.concatenate([hs, a2 + b2n_ref[...]], axis=1), 0.0)
        nrm = jnp.maximum(jnp.sqrt(jnp.sum(h * h, axis=1, keepdims=True)),
                          1e-12)
        o_ref[...] = jnp.dot(h / nrm, wc_ref[...],
                             preferred_element_type=_f32) + bc_ref[...]

    return pl.pallas_call(
        body,
        grid=(N // BM,),
        in_specs=[
            pl.BlockSpec((BM, HID), lambda i: (i, 0)),
            pl.BlockSpec((2, BM, DIN), lambda i: (0, i, 0)),
            pl.BlockSpec((1, BM, DIN), lambda i: (i // 5, i % 5, 0)),
            pl.BlockSpec((HID,), lambda i: (0,)),
            pl.BlockSpec((HID,), lambda i: (0,)),
            pl.BlockSpec((2 * HID, DOUT), lambda i: (0, 0)),
            pl.BlockSpec((DOUT,), lambda i: (0,)),
        ],
        out_specs=pl.BlockSpec((BM, DOUT), lambda i: (i, 0)),
        out_shape=jax.ShapeDtypeStruct((N, DOUT), _f32),
    )(q, agg2, degp, b2s, b2n, Wc, bc)


def kernel(x, edge_index, W1_self, b1_self, W1_neigh, b1_neigh,
           W2_self, b2_self, W2_neigh, b2_neigh, Wc, bc):
    src = edge_index[0].astype(jnp.int32)
    dst = edge_index[1].astype(jnp.int32)
    pad = EPAD - E
    epos = jnp.arange(EPAD, dtype=jnp.int32)
    srcf = jnp.concatenate([src, jnp.arange(pad, dtype=jnp.int32)])
    dst2f = jnp.concatenate([dst, N + jnp.arange(pad, dtype=jnp.int32)
                             % (ACC2 - N)])
    trash1 = HALF + epos % (R1 - HALF)
    dst1 = jnp.stack([
        jnp.where(dst2f < HALF, dst2f, trash1),
        jnp.where((dst2f >= HALF) & (dst2f < N), dst2f - HALF, trash1),
    ]).reshape(2, NROW1, CH1)
    src1 = srcf.reshape(NROW1, CH1)
    src2 = jnp.stack([srcf, srcf + N]).reshape(2, NROW2, CH2)
    dst2 = dst2f.reshape(NROW2, CH2)

    agg_parts, deg_parts = _sc_pass1(x, src1, dst1)
    q, pcat = _tc_phase_b(x, agg_parts, deg_parts,
                          W1_self, b1_self, W1_neigh, b1_neigh,
                          W2_self, W2_neigh)
    agg2 = _sc_pass2(pcat.reshape(2 * N, DIN), src2, dst2)
    return _tc_phase_d(q, agg2, deg_parts, b2_self, b2_neigh, Wc, bc)


# speedup vs baseline: 9.9786x; 1.3479x over previous
import functools

import jax
import jax.numpy as jnp
from jax import lax
from jax.experimental import pallas as pl
from jax.experimental.pallas import tpu as pltpu
from jax.experimental.pallas import tpu_sc as plsc

N = 10000
E = 320000
DIN = 128
HID = 256
DOUT = 64

EPAD = 327680
CH = 128
NROW = EPAD // CH

KW = EPAD // (32 * CH)
SBW = 40

K2 = EPAD // (16 * CH)
SB2 = 32

ACC = 10112
RPS = ACC // 16

_f32 = jnp.float32
_mesh = plsc.VectorSubcoreMesh(core_axis_name="c", subcore_axis_name="s")


def _zero_stripe(zbuf, acc, s):
    @pl.loop(0, RPS // 16)
    def _(k):
        pltpu.sync_copy(zbuf, acc.at[pl.ds(s * RPS + k * 16, 16)])

    pltpu.sync_copy(zbuf.at[pl.ds(0, RPS % 16)],
                    acc.at[pl.ds(s * RPS + (RPS // 16) * 16, RPS % 16)])


def _fill(buf, nrows, val):
    @pl.loop(0, nrows)
    def _(r):
        @pl.loop(0, DIN // 16)
        def _(q):
            buf[r, pl.ds(q * 16, 16)] = val


@functools.partial(
    pl.kernel,
    out_type=jax.ShapeDtypeStruct((2, ACC, DIN), _f32),
    mesh=_mesh,
    scratch_types=[
        pltpu.VMEM((SBW, CH), jnp.int32),
        pltpu.VMEM((CH, DIN), _f32),
        pltpu.VMEM((16, DIN), _f32),
        pltpu.VMEM_SHARED((ACC, DIN), _f32),
    ],
)
def _sc_pass0(dst_hbm, deg_hbm, didx, ones, zbuf, dega):
    c = lax.axis_index("c")
    s = lax.axis_index("s")
    w = c * 16 + s

    _fill(zbuf, 16, jnp.zeros((16,), _f32))
    _fill(ones, CH, jnp.ones((16,), _f32))
    _zero_stripe(zbuf, dega, s)
    plsc.subcore_barrier()

    @pl.loop(0, KW // SBW)
    def _(g):
        pltpu.sync_copy(dst_hbm.at[pl.ds(w * KW + g * SBW, SBW)], didx)

        @pl.loop(0, SBW)
        def _(j):
            pltpu.sync_copy(ones, dega.at[didx.at[j]], add=True)

    plsc.subcore_barrier()
    r0 = s * RPS
    pltpu.sync_copy(dega.at[pl.ds(r0, RPS)], deg_hbm.at[c, pl.ds(r0, RPS)])


@functools.partial(
    pl.kernel,
    out_type=jax.ShapeDtypeStruct((2, ACC, DIN), _f32),
    mesh=_mesh,
    scratch_types=[
        pltpu.VMEM((SBW, CH), jnp.int32),
        pltpu.VMEM((SBW, CH), jnp.int32),
        pltpu.VMEM((CH, DIN), _f32),
        pltpu.VMEM((CH, DIN), _f32),
        pltpu.VMEM((16, DIN), _f32),
        pltpu.VMEM_SHARED((ACC, DIN), _f32),
        pltpu.SemaphoreType.DMA,
        pltpu.SemaphoreType.DMA,
    ],
)
def _sc_pass1(x_hbm, src_hbm, dst_hbm, agg_hbm,
              sidx, didx, rows0, rows1, zbuf, acc, sem0, sem1):
    c = lax.axis_index("c")
    s = lax.axis_index("s")
    w = c * 16 + s

    _fill(zbuf, 16, jnp.zeros((16,), _f32))
    _zero_stripe(zbuf, acc, s)
    plsc.subcore_barrier()

    @pl.loop(0, KW // SBW)
    def _(g):
        b0 = w * KW + g * SBW
        pltpu.sync_copy(src_hbm.at[pl.ds(b0, SBW)], sidx)
        pltpu.sync_copy(dst_hbm.at[pl.ds(b0, SBW)], didx)

        pltpu.async_copy(x_hbm.at[sidx.at[0]], rows0, sem0)

        @pl.loop(0, SBW, step=2)
        def _(j):
            pltpu.make_async_copy(x_hbm.at[sidx.at[j]], rows0, sem0).wait()
            pltpu.async_copy(x_hbm.at[sidx.at[j + 1]], rows1, sem1)
            pltpu.sync_copy(rows0, acc.at[didx.at[j]], add=True)
            pltpu.make_async_copy(x_hbm.at[sidx.at[j + 1]],
                                  rows1, sem1).wait()

            @pl.when(j + 2 < SBW)
            def _():
                pltpu.async_copy(x_hbm.at[sidx.at[j + 2]], rows0, sem0)

            pltpu.sync_copy(rows1, acc.at[didx.at[j + 1]], add=True)

    plsc.subcore_barrier()
    r0 = s * RPS
    pltpu.sync_copy(acc.at[pl.ds(r0, RPS)], agg_hbm.at[c, pl.ds(r0, RPS)])


@functools.partial(
    pl.kernel,
    out_type=jax.ShapeDtypeStruct((2, ACC, DIN), _f32),
    mesh=_mesh,
    scratch_types=[
        pltpu.VMEM((SB2, CH), jnp.int32),
        pltpu.VMEM((SB2, CH), jnp.int32),
        pltpu.VMEM((CH, DIN), _f32),
        pltpu.VMEM((CH, DIN), _f32),
        pltpu.VMEM((16, DIN), _f32),
        pltpu.VMEM_SHARED((ACC, DIN), _f32),
        pltpu.SemaphoreType.DMA,
        pltpu.SemaphoreType.DMA,
    ],
)
def _sc_pass2(p_hbm, src_hbm, dst_hbm, out_hbm,
              sidx, didx, rows0, rows1, zbuf, acc, sem0, sem1):
    c = lax.axis_index("c")
    s = lax.axis_index("s")

    _fill(zbuf, 16, jnp.zeros((16,), _f32))
    _zero_stripe(zbuf, acc, s)
    plsc.subcore_barrier()

    @pl.loop(0, K2 // SB2)
    def _(g):
        b0 = s * K2 + g * SB2
        pltpu.sync_copy(src_hbm.at[c, pl.ds(b0, SB2)], sidx)
        pltpu.sync_copy(dst_hbm.at[pl.ds(b0, SB2)], didx)

        pltpu.async_copy(p_hbm.at[sidx.at[0]], rows0, sem0)

        @pl.loop(0, SB2, step=2)
        def _(j):
            pltpu.make_async_copy(p_hbm.at[sidx.at[j]], rows0, sem0).wait()
            pltpu.async_copy(p_hbm.at[sidx.at[j + 1]], rows1, sem1)
            pltpu.sync_copy(rows0, acc.at[didx.at[j]], add=True)
            pltpu.make_async_copy(p_hbm.at[sidx.at[j + 1]],
                                  rows1, sem1).wait()

            @pl.when(j + 2 < SB2)
            def _():
                pltpu.async_copy(p_hbm.at[sidx.at[j + 2]], rows0, sem0)

            pltpu.sync_copy(rows1, acc.at[didx.at[j + 1]], add=True)

    plsc.subcore_barrier()
    r0 = s * RPS
    pltpu.sync_copy(acc.at[pl.ds(r0, RPS)], out_hbm.at[c, pl.ds(r0, RPS)])


BM = 1000


def _tc_phase_b(x, aggp, degp, W1s, b1s, W1n, b1n, W2s, W2n):
    def body(x_ref, aggp_ref, degp_ref, w1s_ref, b1s_ref, w1n_ref, b1n_ref,
             w2s_ref, w2n_ref, q_ref, pcat_ref):
        deg = degp_ref[0, :, 0:1] + degp_ref[1, :, 0:1]
        inv = 1.0 / jnp.maximum(deg, 1.0)
        agg = (aggp_ref[0] + aggp_ref[1]) * inv
        hs = jnp.dot(x_ref[...], w1s_ref[...],
                     preferred_element_type=_f32) + b1s_ref[...]
        hn = jnp.dot(agg, w1n_ref[...],
                     preferred_element_type=_f32) + b1n_ref[...]
        h = jnp.maximum(jnp.concatenate([hs, hn], axis=1), 0.0)
        q_ref[...] = jnp.dot(h, w2s_ref[...], preferred_element_type=_f32)
        p = jnp.dot(h, w2n_ref[...], preferred_element_type=_f32)
        pcat_ref[0, :, :] = p[:, :DIN]
        pcat_ref[1, :, :] = p[:, DIN:]

    return pl.pallas_call(
        body,
        grid=(N // BM,),
        in_specs=[
            pl.BlockSpec((BM, DIN), lambda i: (i, 0)),
            pl.BlockSpec((2, BM, DIN), lambda i: (0, i, 0)),
            pl.BlockSpec((2, BM, DIN), lambda i: (0, i, 0)),
            pl.BlockSpec((DIN, HID), lambda i: (0, 0)),
            pl.BlockSpec((HID,), lambda i: (0,)),
            pl.BlockSpec((DIN, HID), lambda i: (0, 0)),
            pl.BlockSpec((HID,), lambda i: (0,)),
            pl.BlockSpec((2 * HID, HID), lambda i: (0, 0)),
            pl.BlockSpec((2 * HID, HID), lambda i: (0, 0)),
        ],
        out_specs=[
            pl.BlockSpec((BM, HID), lambda i: (i, 0)),
            pl.BlockSpec((2, BM, DIN), lambda i: (0, i, 0)),
        ],
        out_shape=[jax.ShapeDtypeStruct((N, HID), _f32),
                   jax.ShapeDtypeStruct((2, N, DIN), _f32)],
    )(x, aggp, degp, W1s, b1s, W1n, b1n, W2s, W2n)


def _tc_phase_d(q, agg2, degp, b2s, b2n, Wc, bc):
    def body(q_ref, a2_ref, degp_ref, b2s_ref, b2n_ref, wc_ref, bc_ref,
             o_ref):
        deg = degp_ref[0, :, 0:1] + degp_ref[1, :, 0:1]
        inv = 1.0 / jnp.maximum(deg, 1.0)
        hs = q_ref[...] + b2s_ref[...]
        a2 = jnp.concatenate([a2_ref[0], a2_ref[1]], axis=1) * inv
        h = jnp.maximum(jnp.concatenate([hs, a2 + b2n_ref[...]], axis=1), 0.0)
        nrm = jnp.maximum(jnp.sqrt(jnp.sum(h * h, axis=1, keepdims=True)),
                          1e-12)
        o_ref[...] = jnp.dot(h / nrm, wc_ref[...],
                             preferred_element_type=_f32) + bc_ref[...]

    return pl.pallas_call(
        body,
        grid=(N // BM,),
        in_specs=[
            pl.BlockSpec((BM, HID), lambda i: (i, 0)),
            pl.BlockSpec((2, BM, DIN), lambda i: (0, i, 0)),
            pl.BlockSpec((2, BM, DIN), lambda i: (0, i, 0)),
            pl.BlockSpec((HID,), lambda i: (0,)),
            pl.BlockSpec((HID,), lambda i: (0,)),
            pl.BlockSpec((2 * HID, DOUT), lambda i: (0, 0)),
            pl.BlockSpec((DOUT,), lambda i: (0,)),
        ],
        out_specs=pl.BlockSpec((BM, DOUT), lambda i: (i, 0)),
        out_shape=jax.ShapeDtypeStruct((N, DOUT), _f32),
    )(q, agg2, degp, b2s, b2n, Wc, bc)


def kernel(x, edge_index, W1_self, b1_self, W1_neigh, b1_neigh,
           W2_self, b2_self, W2_neigh, b2_neigh, Wc, bc):
    src = edge_index[0].astype(jnp.int32)
    dst = edge_index[1].astype(jnp.int32)
    pad = EPAD - E
    srcf = jnp.concatenate([src, jnp.arange(pad, dtype=jnp.int32)])
    dstf = jnp.concatenate([dst, N + jnp.arange(pad, dtype=jnp.int32)
                            % (ACC - N)])
    srcp = srcf.reshape(NROW, CH)
    dstp = dstf.reshape(NROW, CH)
    src2 = jnp.stack([srcp, srcp + N])

    deg_parts = _sc_pass0(dstp)
    agg_parts = _sc_pass1(x, srcp, dstp)
    q, pcat = _tc_phase_b(x, agg_parts, deg_parts,
                          W1_self, b1_self, W1_neigh, b1_neigh,
                          W2_self, W2_neigh)
    agg2 = _sc_pass2(pcat.reshape(2 * N, DIN), src2, dstp)
    return _tc_phase_d(q, agg2, deg_parts, b2_self, b2_neigh, Wc, bc)
